# Initial kernel scaffold; baseline (speedup 1.0000x reference)
import jax
import jax.numpy as jnp
from jax.experimental import pallas as pl


def kernel(x, w0, w1, w2, vecs):
    raise NotImplementedError("write your pallas kernel here")


# speedup vs baseline: 1.0167x; 1.0167x over previous
import jax
import jax.numpy as jnp
from jax import lax
from jax.experimental import pallas as pl
from jax.experimental.pallas import tpu as pltpu

_LN_EPS = 1e-5
_ACTION = 6


def _round_up(n, m):
    return ((n + m - 1) // m) * m


def _mlp_kernel(x_ref, w0_ref, w1_ref, w2_ref, vec_ref, o_ref):
    H = w1_ref.shape[0]
    A_pad = w2_ref.shape[1]

    vecs = vec_ref[...]
    b0 = vecs[0:1, :H]
    b1 = vecs[1:2, :H]
    ln_g = vecs[2:3, :H]
    ln_b = vecs[3:4, :H]
    b2 = vecs[4:5, :A_pad]

    x = x_ref[...].astype(jnp.bfloat16)
    h = jnp.dot(x, w0_ref[...], preferred_element_type=jnp.float32) + b0
    h = jnp.dot(h.astype(jnp.bfloat16), w1_ref[...],
                preferred_element_type=jnp.float32) + b1

    mean = jnp.mean(h, axis=-1, keepdims=True)
    var = jnp.mean(jnp.square(h - mean), axis=-1, keepdims=True)
    h = (h - mean) * lax.rsqrt(var + _LN_EPS)
    h = h * ln_g + ln_b
    h = jnp.maximum(h, 0.0)

    out = jnp.dot(h.astype(jnp.bfloat16), w2_ref[...],
                  preferred_element_type=jnp.float32) + b2
    o_ref[...] = out.astype(o_ref.dtype)


def kernel(x, w0, w1, w2, vecs):
    B, S = x.shape
    H = w0.shape[1]
    A_pad = w2.shape[1]

    w0b = w0.astype(jnp.bfloat16)
    w1b = w1.astype(jnp.bfloat16)
    w2b = w2.astype(jnp.bfloat16)
    vecs_f = vecs.astype(jnp.float32)

    if B >= 4096:
        batch_tile = 2048
    elif B >= 512:
        batch_tile = _round_up(B // 2, 8)
    else:
        batch_tile = _round_up(B, 8)
    B_pad = _round_up(B, batch_tile)
    grid = (B_pad // batch_tile,)

    x_p = x if B_pad == B else jnp.zeros((B_pad, S), x.dtype).at[:B].set(x)

    flops = 2 * B_pad * (S * H + H * H + H * A_pad) + 10 * B_pad * H
    bytes_accessed = (x_p.size * 4 + (w0b.size + w1b.size + w2b.size) * 2
                      + vecs_f.size * 4 + B_pad * A_pad * 4)
    cost = pl.CostEstimate(flops=int(flops), transcendentals=int(B_pad),
                           bytes_accessed=int(bytes_accessed))

    def resident(shape):
        return pl.BlockSpec(shape, lambda i: (0,) * len(shape))

    out_padded = pl.pallas_call(
        _mlp_kernel,
        out_shape=jax.ShapeDtypeStruct((B_pad, A_pad), x.dtype),
        grid_spec=pltpu.PrefetchScalarGridSpec(
            num_scalar_prefetch=0,
            grid=grid,
            in_specs=[
                pl.BlockSpec((batch_tile, S), lambda i: (i, 0)),
                resident((S, H)),
                resident((H, H)),
                resident((H, A_pad)),
                resident((8, vecs.shape[1])),
            ],
            out_specs=pl.BlockSpec((batch_tile, A_pad), lambda i: (i, 0)),
        ),
        compiler_params=pltpu.CompilerParams(
            dimension_semantics=("parallel",),
            vmem_limit_bytes=int(48 << 20)),
        cost_estimate=cost,
    )(x_p, w0b, w1b, w2b, vecs_f)

    return out_padded[:B, :_ACTION]


# speedup vs baseline: 1.0685x; 1.0509x over previous
import jax
import jax.numpy as jnp
from jax import lax
from jax.experimental import pallas as pl
from jax.experimental.pallas import tpu as pltpu

_LN_EPS = 1e-5
_ACTION = 6


def _round_up(n, m):
    return ((n + m - 1) // m) * m


def _mlp_kernel(x_ref, w0_ref, w1_ref, w2_ref, vec_ref, o_ref):
    H = w1_ref.shape[0]
    A_pad = w2_ref.shape[1]

    vecs = vec_ref[...]
    b0 = vecs[0:1, :H]
    b1 = vecs[1:2, :H]
    ln_g = vecs[2:3, :H]
    ln_b = vecs[3:4, :H]
    b2 = vecs[4:5, :A_pad]

    x = x_ref[...].astype(jnp.bfloat16)
    h = jnp.dot(x, w0_ref[...], preferred_element_type=jnp.float32) + b0
    h = jnp.dot(h.astype(jnp.bfloat16), w1_ref[...],
                preferred_element_type=jnp.float32) + b1

    inv_h = 1.0 / H
    s1 = jnp.sum(h, axis=-1, keepdims=True)
    s2 = jnp.sum(h * h, axis=-1, keepdims=True)
    mean = s1 * inv_h
    var = s2 * inv_h - mean * mean
    a = lax.rsqrt(var + _LN_EPS)
    h = (h * a - mean * a) * ln_g + ln_b
    h = jnp.maximum(h, 0.0)

    out = jnp.dot(h.astype(jnp.bfloat16), w2_ref[...],
                  preferred_element_type=jnp.float32) + b2
    o_ref[...] = out[:, :o_ref.shape[1]].astype(o_ref.dtype)


def kernel(x, w0, w1, w2, vecs):
    B, S = x.shape
    H = w0.shape[1]
    A_pad = w2.shape[1]

    w0b = w0.astype(jnp.bfloat16)
    w1b = w1.astype(jnp.bfloat16)
    w2b = w2.astype(jnp.bfloat16)
    vecs_f = vecs.astype(jnp.float32)

    if B >= 4096:
        batch_tile = 2048
    elif B >= 512:
        batch_tile = _round_up(B // 2, 8)
    else:
        batch_tile = _round_up(B, 8)
    B_pad = _round_up(B, batch_tile)
    grid = (B_pad // batch_tile,)

    x_p = x if B_pad == B else jnp.zeros((B_pad, S), x.dtype).at[:B].set(x)

    flops = 2 * B_pad * (S * H + H * H + H * A_pad) + 10 * B_pad * H
    bytes_accessed = (x_p.size * 4 + (w0b.size + w1b.size + w2b.size) * 2
                      + vecs_f.size * 4 + B_pad * A_pad * 4)
    cost = pl.CostEstimate(flops=int(flops), transcendentals=int(B_pad),
                           bytes_accessed=int(bytes_accessed))

    def resident(shape):
        return pl.BlockSpec(shape, lambda i: (0,) * len(shape))

    out_padded = pl.pallas_call(
        _mlp_kernel,
        out_shape=jax.ShapeDtypeStruct((B_pad, _ACTION), x.dtype),
        grid_spec=pltpu.PrefetchScalarGridSpec(
            num_scalar_prefetch=0,
            grid=grid,
            in_specs=[
                pl.BlockSpec((batch_tile, S), lambda i: (i, 0)),
                resident((S, H)),
                resident((H, H)),
                resident((H, A_pad)),
                resident((8, vecs.shape[1])),
            ],
            out_specs=pl.BlockSpec((batch_tile, _ACTION), lambda i: (i, 0)),
        ),
        compiler_params=pltpu.CompilerParams(
            dimension_semantics=("parallel",),
            vmem_limit_bytes=int(48 << 20)),
        cost_estimate=cost,
    )(x_p, w0b, w1b, w2b, vecs_f)

    return out_padded[:B]


# speedup vs baseline: 1.0750x; 1.0061x over previous
import jax
import jax.numpy as jnp
from jax import lax
from jax.experimental import pallas as pl
from jax.experimental.pallas import tpu as pltpu

_LN_EPS = 1e-5
_ACTION = 6


def _round_up(n, m):
    return ((n + m - 1) // m) * m


def _mlp_kernel(x_ref, w0_ref, w1_ref, w2_ref, vec_ref, o_ref,
                w0s_ref, w1s_ref, w2s_ref):
    H = w1_ref.shape[0]
    A_pad = w2_ref.shape[1]

    @pl.when(pl.program_id(0) == 0)
    def _cast_weights():
        w0s_ref[...] = w0_ref[...].astype(jnp.bfloat16)
        w1s_ref[...] = w1_ref[...].astype(jnp.bfloat16)
        w2s_ref[...] = w2_ref[...].astype(jnp.bfloat16)

    vecs = vec_ref[...]
    b0 = vecs[0:1, :H]
    b1 = vecs[1:2, :H]
    ln_g = vecs[2:3, :H]
    ln_b = vecs[3:4, :H]
    b2 = vecs[4:5, :A_pad]

    x = x_ref[...].astype(jnp.bfloat16)
    h = jnp.dot(x, w0s_ref[...], preferred_element_type=jnp.float32) + b0
    h = jnp.dot(h.astype(jnp.bfloat16), w1s_ref[...],
                preferred_element_type=jnp.float32) + b1

    inv_h = 1.0 / H
    s1 = jnp.sum(h, axis=-1, keepdims=True)
    s2 = jnp.sum(h * h, axis=-1, keepdims=True)
    mean = s1 * inv_h
    var = s2 * inv_h - mean * mean
    a = lax.rsqrt(var + _LN_EPS)
    h = (h * a - mean * a) * ln_g + ln_b
    h = jnp.maximum(h, 0.0)

    out = jnp.dot(h.astype(jnp.bfloat16), w2s_ref[...],
                  preferred_element_type=jnp.float32) + b2
    o_ref[...] = out[:, :o_ref.shape[1]].astype(o_ref.dtype)


def kernel(x, w0, w1, w2, vecs):
    B, S = x.shape
    H = w0.shape[1]
    A_pad = w2.shape[1]
    vecs_f = vecs.astype(jnp.float32)

    if B >= 4096:
        batch_tile = 2048
    elif B >= 512:
        batch_tile = _round_up(B // 2, 8)
    else:
        batch_tile = _round_up(B, 8)
    B_pad = _round_up(B, batch_tile)
    grid = (B_pad // batch_tile,)

    x_p = x if B_pad == B else jnp.zeros((B_pad, S), x.dtype).at[:B].set(x)

    flops = 2 * B_pad * (S * H + H * H + H * A_pad) + 10 * B_pad * H
    bytes_accessed = (x_p.size * 4 + (w0.size + w1.size + w2.size) * 4
                      + vecs_f.size * 4 + B_pad * _ACTION * 4)
    cost = pl.CostEstimate(flops=int(flops), transcendentals=int(B_pad),
                           bytes_accessed=int(bytes_accessed))

    def resident(shape):
        return pl.BlockSpec(shape, lambda i: (0,) * len(shape))

    out = pl.pallas_call(
        _mlp_kernel,
        out_shape=jax.ShapeDtypeStruct((B_pad, _ACTION), x.dtype),
        grid_spec=pltpu.PrefetchScalarGridSpec(
            num_scalar_prefetch=0,
            grid=grid,
            in_specs=[
                pl.BlockSpec((batch_tile, S), lambda i: (i, 0)),
                resident((S, H)),
                resident((H, H)),
                resident((H, A_pad)),
                resident((8, vecs.shape[1])),
            ],
            out_specs=pl.BlockSpec((batch_tile, _ACTION), lambda i: (i, 0)),
            scratch_shapes=[
                pltpu.VMEM((S, H), jnp.bfloat16),
                pltpu.VMEM((H, H), jnp.bfloat16),
                pltpu.VMEM((H, A_pad), jnp.bfloat16),
            ],
        ),
        compiler_params=pltpu.CompilerParams(
            dimension_semantics=("arbitrary",),
            vmem_limit_bytes=int(48 << 20)),
        cost_estimate=cost,
    )(x_p, w0, w1, w2, vecs_f)

    return out[:B]


# speedup vs baseline: 1.5608x; 1.4519x over previous
import jax
import jax.numpy as jnp
from jax import lax
from jax.experimental import pallas as pl
from jax.experimental.pallas import tpu as pltpu

_LN_EPS = 1e-5
_ACTION = 6


def _round_up(n, m):
    return ((n + m - 1) // m) * m


def _mlp_kernel(x_ref, w0_ref, w1_ref, w2_ref, vec_ref, o_ref,
                w01s_ref, w2s_ref, b01s_ref):
    H = w1_ref.shape[0]
    A_pad = w2_ref.shape[1]

    vecs = vec_ref[...]
    ln_g = vecs[2:3, :H]
    ln_b = vecs[3:4, :H]
    b2 = vecs[4:5, :A_pad]

    @pl.when(pl.program_id(0) == 0)
    def _compose_weights():
        w0b = w0_ref[...].astype(jnp.bfloat16)
        w1b = w1_ref[...].astype(jnp.bfloat16)
        w01 = jnp.dot(w0b, w1b, preferred_element_type=jnp.float32)
        w01s_ref[...] = w01.astype(jnp.bfloat16)
        w2s_ref[...] = w2_ref[...].astype(jnp.bfloat16)
        b0 = vecs[0:1, :H]
        b1 = vecs[1:2, :H]
        b01 = jnp.dot(b0.astype(jnp.bfloat16), w1b,
                      preferred_element_type=jnp.float32) + b1
        b01s_ref[...] = jnp.broadcast_to(b01, b01s_ref.shape)

    x = x_ref[...].astype(jnp.bfloat16)
    h = jnp.dot(x, w01s_ref[...],
                preferred_element_type=jnp.float32) + b01s_ref[0:1, :]

    inv_h = 1.0 / H
    s1 = jnp.sum(h, axis=-1, keepdims=True)
    s2 = jnp.sum(h * h, axis=-1, keepdims=True)
    mean = s1 * inv_h
    var = s2 * inv_h - mean * mean
    a = lax.rsqrt(var + _LN_EPS)
    h = (h * a - mean * a) * ln_g + ln_b
    h = jnp.maximum(h, 0.0)

    out = jnp.dot(h.astype(jnp.bfloat16), w2s_ref[...],
                  preferred_element_type=jnp.float32) + b2
    o_ref[...] = out[:, :o_ref.shape[1]].astype(o_ref.dtype)


def kernel(x, w0, w1, w2, vecs):
    B, S = x.shape
    H = w0.shape[1]
    A_pad = w2.shape[1]
    vecs_f = vecs.astype(jnp.float32)

    if B >= 4096:
        batch_tile = 2048
    elif B >= 512:
        batch_tile = _round_up(B // 2, 8)
    else:
        batch_tile = _round_up(B, 8)
    B_pad = _round_up(B, batch_tile)
    grid = (B_pad // batch_tile,)

    x_p = x if B_pad == B else jnp.zeros((B_pad, S), x.dtype).at[:B].set(x)

    flops = 2 * B_pad * (S * H + H * A_pad) + 10 * B_pad * H
    bytes_accessed = (x_p.size * 4 + (w0.size + w1.size + w2.size) * 4
                      + vecs_f.size * 4 + B_pad * _ACTION * 4)
    cost = pl.CostEstimate(flops=int(flops), transcendentals=int(B_pad),
                           bytes_accessed=int(bytes_accessed))

    def resident(shape):
        return pl.BlockSpec(shape, lambda i: (0,) * len(shape))

    out = pl.pallas_call(
        _mlp_kernel,
        out_shape=jax.ShapeDtypeStruct((B_pad, _ACTION), x.dtype),
        grid_spec=pltpu.PrefetchScalarGridSpec(
            num_scalar_prefetch=0,
            grid=grid,
            in_specs=[
                pl.BlockSpec((batch_tile, S), lambda i: (i, 0)),
                resident((S, H)),
                resident((H, H)),
                resident((H, A_pad)),
                resident((8, vecs.shape[1])),
            ],
            out_specs=pl.BlockSpec((batch_tile, _ACTION), lambda i: (i, 0)),
            scratch_shapes=[
                pltpu.VMEM((S, H), jnp.bfloat16),
                pltpu.VMEM((H, A_pad), jnp.bfloat16),
                pltpu.VMEM((8, H), jnp.float32),
            ],
        ),
        compiler_params=pltpu.CompilerParams(
            dimension_semantics=("arbitrary",),
            vmem_limit_bytes=int(48 << 20)),
        cost_estimate=cost,
    )(x_p, w0, w1, w2, vecs_f)

    return out[:B]


# speedup vs baseline: 1.7071x; 1.0937x over previous
import jax
import jax.numpy as jnp
from jax import lax
from jax.experimental import pallas as pl
from jax.experimental.pallas import tpu as pltpu

_LN_EPS = 1e-5
_ACTION = 6


def _round_up(n, m):
    return ((n + m - 1) // m) * m


def _mlp_kernel(x_ref, w0_ref, w1_ref, w2_ref, vec_ref, o_ref,
                w01s_ref, w2s_ref, b01s_ref):
    H = w1_ref.shape[0]
    A_pad = w2_ref.shape[1]

    vecs = vec_ref[...]
    ln_g = vecs[2:3, :H]
    ln_b = vecs[3:4, :H]
    b2 = vecs[4:5, :A_pad]

    @pl.when(pl.program_id(0) == 0)
    def _compose_weights():
        w0b = w0_ref[...].astype(jnp.bfloat16)
        w1b = w1_ref[...].astype(jnp.bfloat16)
        w01 = jnp.dot(w0b, w1b, preferred_element_type=jnp.float32)
        w01s_ref[...] = w01
        w2s_ref[...] = w2_ref[...]
        b0 = vecs[0:1, :H]
        b1 = vecs[1:2, :H]
        b01 = jnp.dot(b0.astype(jnp.bfloat16), w1b,
                      preferred_element_type=jnp.float32) + b1
        b01s_ref[...] = jnp.broadcast_to(b01, b01s_ref.shape)

    TB = x_ref.shape[0]
    n_chunks = 4 if TB >= 2048 else (2 if TB >= 1024 else 1)
    C = TB // n_chunks
    inv_h = 1.0 / H
    for c in range(n_chunks):
        rows = pl.ds(c * C, C)
        h = jnp.dot(x_ref[rows, :], w01s_ref[...],
                    preferred_element_type=jnp.float32) + b01s_ref[0:1, :]

        s1 = jnp.sum(h, axis=-1, keepdims=True)
        s2 = jnp.sum(h * h, axis=-1, keepdims=True)
        mean = s1 * inv_h
        var = s2 * inv_h - mean * mean
        a = lax.rsqrt(var + _LN_EPS)
        h = (h * a - mean * a) * ln_g + ln_b
        h = jnp.maximum(h, 0.0)

        out = jnp.dot(h, w2s_ref[...],
                      preferred_element_type=jnp.float32) + b2
        o_ref[rows, :] = out[:, :o_ref.shape[1]].astype(o_ref.dtype)


def kernel(x, w0, w1, w2, vecs):
    B, S = x.shape
    H = w0.shape[1]
    A_pad = w2.shape[1]
    vecs_f = vecs.astype(jnp.float32)

    if B >= 4096:
        batch_tile = 2048
    elif B >= 512:
        batch_tile = _round_up(B // 2, 8)
    else:
        batch_tile = _round_up(B, 8)
    B_pad = _round_up(B, batch_tile)
    grid = (B_pad // batch_tile,)

    x_p = x if B_pad == B else jnp.zeros((B_pad, S), x.dtype).at[:B].set(x)

    flops = 2 * B_pad * (S * H + H * A_pad) + 10 * B_pad * H
    bytes_accessed = (x_p.size * 4 + (w0.size + w1.size + w2.size) * 4
                      + vecs_f.size * 4 + B_pad * _ACTION * 4)
    cost = pl.CostEstimate(flops=int(flops), transcendentals=int(B_pad),
                           bytes_accessed=int(bytes_accessed))

    def resident(shape):
        return pl.BlockSpec(shape, lambda i: (0,) * len(shape))

    out = pl.pallas_call(
        _mlp_kernel,
        out_shape=jax.ShapeDtypeStruct((B_pad, _ACTION), x.dtype),
        grid_spec=pltpu.PrefetchScalarGridSpec(
            num_scalar_prefetch=0,
            grid=grid,
            in_specs=[
                pl.BlockSpec((batch_tile, S), lambda i: (i, 0)),
                resident((S, H)),
                resident((H, H)),
                resident((H, A_pad)),
                resident((8, vecs.shape[1])),
            ],
            out_specs=pl.BlockSpec((batch_tile, _ACTION), lambda i: (i, 0)),
            scratch_shapes=[
                pltpu.VMEM((S, H), jnp.float32),
                pltpu.VMEM((H, A_pad), jnp.float32),
                pltpu.VMEM((8, H), jnp.float32),
            ],
        ),
        compiler_params=pltpu.CompilerParams(
            dimension_semantics=("arbitrary",),
            vmem_limit_bytes=int(48 << 20)),
        cost_estimate=cost,
    )(x_p, w0, w1, w2, vecs_f)

    return out[:B]


# speedup vs baseline: 1.8289x; 1.0713x over previous
import jax
import jax.numpy as jnp
from jax import lax
from jax.experimental import pallas as pl
from jax.experimental.pallas import tpu as pltpu

_LN_EPS = 1e-5
_ACTION = 6


def _round_up(n, m):
    return ((n + m - 1) // m) * m


def _mlp_kernel(x_ref, w0_ref, w1_ref, w2_ref, vec_ref, o_ref,
                w01s_ref, w2s_ref, b01s_ref):
    H = w1_ref.shape[0]
    A_pad = w2_ref.shape[1]

    vecs = vec_ref[...]
    ln_g = vecs[2:3, :H]
    ln_b = vecs[3:4, :H]
    b2 = vecs[4:5, :A_pad]

    @pl.when(pl.program_id(0) == 0)
    def _compose_weights():
        w0b = w0_ref[...].astype(jnp.bfloat16)
        w1b = w1_ref[...].astype(jnp.bfloat16)
        w01 = jnp.dot(w0b, w1b, preferred_element_type=jnp.float32)
        w01s_ref[...] = w01
        w2s_ref[...] = w2_ref[...]
        b0 = vecs[0:1, :H]
        b1 = vecs[1:2, :H]
        b01 = jnp.dot(b0.astype(jnp.bfloat16), w1b,
                      preferred_element_type=jnp.float32) + b1
        b01s_ref[...] = jnp.broadcast_to(b01, b01s_ref.shape)

    TB = x_ref.shape[0]
    n_chunks = max(1, TB // 512)
    C = TB // n_chunks
    inv_h = 1.0 / H
    for c in range(n_chunks):
        rows = pl.ds(c * C, C)
        h = jnp.dot(x_ref[rows, :], w01s_ref[...],
                    preferred_element_type=jnp.float32) + b01s_ref[0:1, :]

        s1 = jnp.sum(h, axis=-1, keepdims=True)
        s2 = jnp.sum(h * h, axis=-1, keepdims=True)
        mean = s1 * inv_h
        var = s2 * inv_h - mean * mean
        a = lax.rsqrt(var + _LN_EPS)
        h = (h * a - mean * a) * ln_g + ln_b
        h = jnp.maximum(h, 0.0)

        out = jnp.dot(h, w2s_ref[...],
                      preferred_element_type=jnp.float32) + b2
        o_ref[rows, :] = out[:, :o_ref.shape[1]].astype(o_ref.dtype)


def kernel(x, w0, w1, w2, vecs):
    B, S = x.shape
    H = w0.shape[1]
    A_pad = w2.shape[1]
    vecs_f = vecs.astype(jnp.float32)

    if B >= 4096:
        batch_tile = 4096
    elif B >= 512:
        batch_tile = _round_up(B // 2, 8)
    else:
        batch_tile = _round_up(B, 8)
    B_pad = _round_up(B, batch_tile)
    grid = (B_pad // batch_tile,)

    x_p = x if B_pad == B else jnp.zeros((B_pad, S), x.dtype).at[:B].set(x)

    flops = 2 * B_pad * (S * H + H * A_pad) + 10 * B_pad * H
    bytes_accessed = (x_p.size * 4 + (w0.size + w1.size + w2.size) * 4
                      + vecs_f.size * 4 + B_pad * _ACTION * 4)
    cost = pl.CostEstimate(flops=int(flops), transcendentals=int(B_pad),
                           bytes_accessed=int(bytes_accessed))

    def resident(shape):
        return pl.BlockSpec(shape, lambda i: (0,) * len(shape))

    out = pl.pallas_call(
        _mlp_kernel,
        out_shape=jax.ShapeDtypeStruct((B_pad, _ACTION), x.dtype),
        grid_spec=pltpu.PrefetchScalarGridSpec(
            num_scalar_prefetch=0,
            grid=grid,
            in_specs=[
                pl.BlockSpec((batch_tile, S), lambda i: (i, 0)),
                resident((S, H)),
                resident((H, H)),
                resident((H, A_pad)),
                resident((8, vecs.shape[1])),
            ],
            out_specs=pl.BlockSpec((batch_tile, _ACTION), lambda i: (i, 0)),
            scratch_shapes=[
                pltpu.VMEM((S, H), jnp.float32),
                pltpu.VMEM((H, A_pad), jnp.float32),
                pltpu.VMEM((8, H), jnp.float32),
            ],
        ),
        compiler_params=pltpu.CompilerParams(
            dimension_semantics=("arbitrary",),
            vmem_limit_bytes=int(56 << 20)),
        cost_estimate=cost,
    )(x_p, w0, w1, w2, vecs_f)

    return out[:B]
